# baseline (speedup 1.0000x reference)
import functools
import math

import jax
import jax.numpy as jnp
from jax import lax
from jax.experimental import pallas as pl
from jax.experimental.pallas import tpu as pltpu
from jax.experimental.pallas import tpu_sc as plsc

SEQ, B, DIM = 4096, 8, 1024
LANES = 16
CHUNK = 8
N_WORKERS = 32
ROWS_PER_WORKER = SEQ // N_WORKERS
N_CHUNKS = ROWS_PER_WORKER // CHUNK


def _sc_body(emb_hbm, pe_hbm, out_hbm, ebuf, pbuf, *, scale):
    wid = lax.axis_index("s") * 2 + lax.axis_index("c")
    base = wid * ROWS_PER_WORKER

    def chunk_body(g):
        r0 = base + g * CHUNK
        pltpu.sync_copy(emb_hbm.at[pl.ds(r0, CHUNK)], ebuf)
        pltpu.sync_copy(pe_hbm.at[pl.ds(r0, CHUNK)], pbuf)

        def row_body(s):
            for bb in range(B):
                for k in range(DIM // LANES):
                    sl = pl.ds(k * LANES, LANES)
                    ebuf[s, bb, sl] = ebuf[s, bb, sl] * scale + pbuf[s, 0, sl]

        pl.loop(0, CHUNK)(row_body)
        pltpu.sync_copy(ebuf, out_hbm.at[pl.ds(r0, CHUNK)])

    pl.loop(0, N_CHUNKS)(chunk_body)


def kernel(emb, src_org, pe):
    del src_org
    seq, b, dim = emb.shape
    scale = math.sqrt(pe.shape[-1])

    mesh = plsc.VectorSubcoreMesh(core_axis_name="c", subcore_axis_name="s")
    sc_call = functools.partial(
        pl.kernel,
        mesh=mesh,
        out_type=jax.ShapeDtypeStruct((seq, b, dim), emb.dtype),
        scratch_types=[
            pltpu.VMEM((CHUNK, b, dim), jnp.float32),
            pltpu.VMEM((CHUNK, 1, dim), jnp.float32),
        ],
    )(functools.partial(_sc_body, scale=scale))
    return sc_call(emb, pe[:seq])
